# Initial kernel scaffold; baseline (speedup 1.0000x reference)
import jax
import jax.numpy as jnp
from jax.experimental import pallas as pl


def kernel(user, item, theta_w, a_w, b_w, c_w):
    raise NotImplementedError("write your pallas kernel here")


# speedup vs baseline: 1.2228x; 1.2228x over previous
import functools

import jax
import jax.numpy as jnp
from jax import lax
from jax.experimental import pallas as pl
from jax.experimental.pallas import tpu as pltpu
from jax.experimental.pallas import tpu_sc as plsc

BATCH = 16384
NC, NS, L = 2, 16, 16
NW = NC * NS
BPW = BATCH // NW
D_CONST = 1.702


def _irt_body(user_hbm, item_hbm, theta_hbm, a_hbm, b_hbm, c_hbm, out_hbm,
              uidx, iidx, tv, av, bv, cv, ov, sem):
    wid = lax.axis_index("s") * NC + lax.axis_index("c")
    base = wid * BPW
    pltpu.sync_copy(user_hbm.at[pl.ds(base, BPW)], uidx)
    pltpu.sync_copy(item_hbm.at[pl.ds(base, BPW)], iidx)
    c1 = pltpu.async_copy(theta_hbm.at[uidx], tv, sem)
    c2 = pltpu.async_copy(a_hbm.at[iidx], av, sem)
    c3 = pltpu.async_copy(b_hbm.at[iidx], bv, sem)
    c4 = pltpu.async_copy(c_hbm.at[iidx], cv, sem)
    c1.wait(); c2.wait(); c3.wait(); c4.wait()
    for j in range(BPW // L):
        sl = pl.ds(j * L, L)
        t = tv[sl]
        a = jnp.maximum(av[sl], 0.001)
        b = bv[sl]
        c = jnp.clip(cv[sl], 0.0, 1.0)
        sig = 1.0 / (1.0 + jnp.exp(-D_CONST * a * (t - b)))
        ov[sl] = c + (1.0 - c) * sig
    pltpu.sync_copy(ov, out_hbm.at[pl.ds(base, BPW)])


def kernel(user, item, theta_w, a_w, b_w, c_w):
    user = user.astype(jnp.int32)
    item = item.astype(jnp.int32)
    theta_flat = theta_w.reshape(-1)
    a_flat = a_w.reshape(-1)
    b_flat = b_w.reshape(-1)
    c_flat = c_w.reshape(-1)
    mesh = plsc.VectorSubcoreMesh(core_axis_name="c", subcore_axis_name="s")
    run = pl.kernel(
        _irt_body,
        mesh=mesh,
        out_type=jax.ShapeDtypeStruct((BATCH,), jnp.float32),
        scratch_types=[
            pltpu.VMEM((BPW,), jnp.int32),
            pltpu.VMEM((BPW,), jnp.int32),
            pltpu.VMEM((BPW,), jnp.float32),
            pltpu.VMEM((BPW,), jnp.float32),
            pltpu.VMEM((BPW,), jnp.float32),
            pltpu.VMEM((BPW,), jnp.float32),
            pltpu.VMEM((BPW,), jnp.float32),
            pltpu.SemaphoreType.DMA,
        ],
    )
    return run(user, item, theta_flat, a_flat, b_flat, c_flat)
